# baseline (speedup 1.0000x reference)
import jax
import jax.numpy as jnp
from jax import lax
from jax.experimental import pallas as pl
from jax.experimental.pallas import tpu as pltpu
from jax.experimental.pallas import tpu_sc as plsc

N_NODES = 10000
N_EDGES = 320000
D = 128

NC = 2
NS = 16
CHUNK = 128
NCHUNKS = N_EDGES // CHUNK
CHUNKS_PER_CORE = NCHUNKS // NC
CPT = CHUNKS_PER_CORE // NS
NLEFT = CHUNKS_PER_CORE - NS * CPT
ROWS_A = 632
ROWS_B = N_NODES - (NS - 1) * ROWS_A


def _fill_vmem_rows(ref, nrows, ncols, value):
  v16 = jnp.full((16,), value, jnp.float32)

  def body(i, carry):
    for k in range(ncols // 16):
      ref[i, pl.ds(k * 16, 16)] = v16
    return carry

  lax.fori_loop(0, nrows, body, 0)


def _make_sc_agg(with_cnt):
  mesh = plsc.VectorSubcoreMesh(core_axis_name="c", subcore_axis_name="s")

  out_type = [jax.ShapeDtypeStruct((NC * N_NODES, D), jnp.float32)]
  if with_cnt:
    out_type.append(jax.ShapeDtypeStruct((NC * N_NODES, D), jnp.float32))

  scratch = [
      pltpu.VMEM((2, CHUNK), jnp.int32),
      pltpu.VMEM((2, CHUNK), jnp.int32),
      pltpu.VMEM((2, CHUNK), jnp.int32),
      pltpu.VMEM((CHUNK,), jnp.int32),
      pltpu.VMEM((CHUNK,), jnp.int32),
      pltpu.VMEM((CHUNK,), jnp.int32),
      pltpu.VMEM((CHUNK, D), jnp.float32),
      pltpu.VMEM((CHUNK, D), jnp.float32),
      pltpu.VMEM((CHUNK, D), jnp.float32),
      pltpu.VMEM_SHARED((N_NODES, D), jnp.float32),
      pltpu.SemaphoreType.DMA,
      pltpu.SemaphoreType.DMA,
      pltpu.SemaphoreType.DMA,
      pltpu.SemaphoreType.DMA,
      pltpu.SemaphoreType.DMA,
      pltpu.SemaphoreType.DMA,
      pltpu.SemaphoreType.DMA,
      pltpu.SemaphoreType.DMA,
      pltpu.SemaphoreType.DMA,
  ]

  def body(h_hbm, ep_hbm, agg_out, *rest):
    if with_cnt:
      (cnt_out, sd0, sd1, sd2, db0, db1, db2, r0v, r1v, r2v, acc_sh,
       is0, is1, is2, gs0, gs1, gs2, ss0, ss1, ss2) = rest
    else:
      (sd0, sd1, sd2, db0, db1, db2, r0v, r1v, r2v, acc_sh,
       is0, is1, is2, gs0, gs1, gs2, ss0, ss1, ss2) = rest
    sds = (sd0, sd1, sd2)
    dbs = (db0, db1, db2)
    rvs = (r0v, r1v, r2v)
    iss = (is0, is1, is2)
    gss = (gs0, gs1, gs2)
    sss = (ss0, ss1, ss2)

    c = lax.axis_index("c")
    s = lax.axis_index("s")
    row0 = pl.multiple_of(s * ROWS_A, 8)
    out0 = pl.multiple_of(c * N_NODES + s * ROWS_A, 8)
    cbase = c * CHUNKS_PER_CORE + s * CPT
    clast = cbase + CPT - 1
    xtra = jnp.where(s < NLEFT, 1, 0)
    cx = c * CHUNKS_PER_CORE + NS * CPT + s

    def idx_load(jj, sd, sem):
      pltpu.async_copy(ep_hbm.at[jj], sd, sem)

    def idx_wait(sd, sem):
      pltpu.make_async_copy(ep_hbm.at[0], sd, sem).wait()

    def g_start(sd, rv, sem):
      pltpu.async_copy(h_hbm.at[sd.at[0]], rv, sem)

    def g_wait(sd, rv, sem):
      pltpu.make_async_copy(h_hbm.at[sd.at[0]], rv, sem).wait()

    def copy_dst(sd, db):
      for k in range(CHUNK // 16):
        db[pl.ds(k * 16, 16)] = sd[1, pl.ds(k * 16, 16)]

    def ss_start(rv, db, sem):
      pltpu.async_copy(rv, acc_sh.at[db], sem, add=True)

    def ss_wait(rv, db, sem):
      pltpu.make_async_copy(rv, acc_sh.at[db], sem).wait()

    def zero_stripe(nrows):
      nfull = nrows // CHUNK
      rem = nrows - nfull * CHUNK

      def zinit(j, carry):
        off = pl.multiple_of(row0 + j * CHUNK, 8)
        pltpu.async_copy(r0v, acc_sh.at[pl.ds(off, CHUNK)], gs0)
        return carry

      lax.fori_loop(0, nfull, zinit, 0)
      offr = pl.multiple_of(row0 + nfull * CHUNK, 8)
      pltpu.async_copy(r0v.at[pl.ds(0, rem)], acc_sh.at[pl.ds(offr, rem)],
                       gs0)

      def zdrain(j, carry):
        pltpu.make_async_copy(r0v, acc_sh.at[pl.ds(row0, CHUNK)], gs0).wait()
        return carry

      lax.fori_loop(0, nfull, zdrain, 0)
      pltpu.make_async_copy(r0v.at[pl.ds(0, rem)],
                            acc_sh.at[pl.ds(row0, rem)], gs0).wait()

    def zero_acc():
      @pl.when(s < NS - 1)
      def _():
        zero_stripe(ROWS_A)

      @pl.when(s == NS - 1)
      def _():
        zero_stripe(ROWS_B)

    def copy_out(dst):
      @pl.when(s < NS - 1)
      def _():
        pltpu.sync_copy(acc_sh.at[pl.ds(row0, ROWS_A)],
                        dst.at[pl.ds(out0, ROWS_A)])

      @pl.when(s == NS - 1)
      def _():
        pltpu.sync_copy(acc_sh.at[pl.ds(row0, ROWS_B)],
                        dst.at[pl.ds(out0, ROWS_B)])

    if with_cnt:
      _fill_vmem_rows(r0v, CHUNK, D, 0.0)
      zero_acc()
      _fill_vmem_rows(r0v, CHUNK, D, 1.0)
      plsc.subcore_barrier()

      for i in range(3):
        idx_load(cbase + i, sds[i], iss[i])
      for i in range(3):
        idx_wait(sds[i], iss[i])
        copy_dst(sds[i], dbs[i])
        idx_load(cbase + i + 3, sds[i], iss[i])
        ss_start(r0v, dbs[i], sss[i])

      def ctriple(p, carry):
        j = cbase + 3 + 3 * p
        for i in range(3):
          ss_wait(r0v, dbs[i], sss[i])
          idx_wait(sds[i], iss[i])
          copy_dst(sds[i], dbs[i])
          idx_load(jnp.minimum(j + i + 3, clast), sds[i], iss[i])
          ss_start(r0v, dbs[i], sss[i])
        return carry

      lax.fori_loop(0, CPT // 3 - 1, ctriple, 0)
      for i in range(3):
        ss_wait(r0v, dbs[i], sss[i])
        idx_wait(sds[i], iss[i])

      def cleft(j, carry):
        idx_load(cx, sd0, is0)
        idx_wait(sd0, is0)
        pltpu.sync_copy(r0v, acc_sh.at[sd0.at[1]], add=True)
        return carry

      lax.fori_loop(0, xtra, cleft, 0)
      plsc.subcore_barrier()
      copy_out(cnt_out)
      plsc.subcore_barrier()

    _fill_vmem_rows(r0v, CHUNK, D, 0.0)
    zero_acc()
    plsc.subcore_barrier()

    for i in range(3):
      idx_load(cbase + i, sds[i], iss[i])
      idx_wait(sds[i], iss[i])
      g_start(sds[i], rvs[i], gss[i])
    for i in range(3):
      g_wait(sds[i], rvs[i], gss[i])
      copy_dst(sds[i], dbs[i])
      idx_load(cbase + i + 3, sds[i], iss[i])
      ss_start(rvs[i], dbs[i], sss[i])
    ss_wait(rvs[0], dbs[0], sss[0])
    idx_wait(sds[0], iss[0])
    g_start(sds[0], rvs[0], gss[0])

    def triple(p, carry):
      j = cbase + 3 + 3 * p
      for i in range(3):
        i1 = (i + 1) % 3
        g_wait(sds[i], rvs[i], gss[i])
        copy_dst(sds[i], dbs[i])
        idx_load(jnp.minimum(j + i + 3, clast), sds[i], iss[i])
        ss_start(rvs[i], dbs[i], sss[i])
        ss_wait(rvs[i1], dbs[i1], sss[i1])
        idx_wait(sds[i1], iss[i1])
        g_start(sds[i1], rvs[i1], gss[i1])
      return carry

    lax.fori_loop(0, CPT // 3 - 1, triple, 0)
    ss_wait(rvs[1], dbs[1], sss[1])
    ss_wait(rvs[2], dbs[2], sss[2])
    g_wait(sds[0], rvs[0], gss[0])
    idx_wait(sds[1], iss[1])
    idx_wait(sds[2], iss[2])

    def aleft(j, carry):
      idx_load(cx, sd0, is0)
      idx_wait(sd0, is0)
      g_start(sd0, r0v, gs0)
      g_wait(sd0, r0v, gs0)
      pltpu.sync_copy(r0v, acc_sh.at[sd0.at[1]], add=True)
      return carry

    lax.fori_loop(0, xtra, aleft, 0)

    plsc.subcore_barrier()
    copy_out(agg_out)

  return pl.kernel(body, out_type=tuple(out_type), mesh=mesh,
                   scratch_types=scratch)


_sc_agg_cnt = _make_sc_agg(True)
_sc_agg = _make_sc_agg(False)


def _make_tc_combine(relu):
  R = 1000

  def body(agg0, agg1, cnt0, cnt1, h, wl, wr, b, out):
    cnt = cnt0[:, 0:1] + cnt1[:, 0:1]
    inv = 1.0 / jnp.maximum(cnt, 1.0)
    agg = (agg0[...] + agg1[...]) * inv
    acc = jnp.dot(agg, wl[...], preferred_element_type=jnp.float32)
    acc = acc + jnp.dot(h[...], wr[...], preferred_element_type=jnp.float32)
    acc = acc + b[...]
    out[...] = jnp.maximum(acc, 0.0) if relu else acc

  row_spec = pl.BlockSpec((R, D), lambda i: (i, 0))
  w_spec = pl.BlockSpec((D, D), lambda i: (0, 0))
  return pl.pallas_call(
      body,
      grid=(N_NODES // R,),
      in_specs=[row_spec, row_spec, row_spec, row_spec, row_spec,
                w_spec, w_spec, pl.BlockSpec((1, D), lambda i: (0, 0))],
      out_specs=row_spec,
      out_shape=jax.ShapeDtypeStruct((N_NODES, D), jnp.float32),
  )


_tc_combine_relu = _make_tc_combine(True)
_tc_combine = _make_tc_combine(False)


def kernel(x, edge_index, W1l, W1r, b1, W2l, W2r, b2, W3l, W3r, b3):
  src = edge_index[0]
  dst = edge_index[1]
  epairs = jnp.stack(
      [src.reshape(NCHUNKS, CHUNK), dst.reshape(NCHUNKS, CHUNK)], axis=1)

  aggp, cntp = _sc_agg_cnt(x, epairs)
  cnt0, cnt1 = cntp[:N_NODES], cntp[N_NODES:]

  def layer(h, aggp, Wl, Wr, b, relu):
    f = _tc_combine_relu if relu else _tc_combine
    return f(aggp[:N_NODES], aggp[N_NODES:], cnt0, cnt1, h,
             Wl, Wr, b.reshape(1, D))

  h1 = layer(x, aggp, W1l, W1r, b1, True)
  aggp2, = _sc_agg(h1, epairs)
  h2 = layer(h1, aggp2, W2l, W2r, b2, True)
  aggp3, = _sc_agg(h2, epairs)
  h3 = layer(h2, aggp3, W3l, W3r, b3, False)
  return h3


# speedup vs baseline: 1.2200x; 1.2200x over previous
import jax
import jax.numpy as jnp
from jax import lax
from jax.experimental import pallas as pl
from jax.experimental.pallas import tpu as pltpu
from jax.experimental.pallas import tpu_sc as plsc

N_NODES = 10000
N_EDGES = 320000
D = 128

NC = 2
NS = 16
CHUNK = 128
NCHUNKS = N_EDGES // CHUNK
CHUNKS_PER_CORE = NCHUNKS // NC
CPT = CHUNKS_PER_CORE // NS
NLEFT = CHUNKS_PER_CORE - NS * CPT
ROWS_A = 632
ROWS_B = N_NODES - (NS - 1) * ROWS_A


def _fill_vmem_rows(ref, nrows, ncols, value):
  v16 = jnp.full((16,), value, jnp.float32)

  def body(i, carry):
    for k in range(ncols // 16):
      ref[i, pl.ds(k * 16, 16)] = v16
    return carry

  lax.fori_loop(0, nrows, body, 0)


def _make_sc_agg(with_cnt):
  mesh = plsc.VectorSubcoreMesh(core_axis_name="c", subcore_axis_name="s")

  out_type = [jax.ShapeDtypeStruct((NC * N_NODES, D), jnp.float32)]
  if with_cnt:
    out_type.append(jax.ShapeDtypeStruct((NC * N_NODES, D), jnp.float32))

  scratch = [
      pltpu.VMEM((2, CHUNK), jnp.int32),
      pltpu.VMEM((2, CHUNK), jnp.int32),
      pltpu.VMEM((2, CHUNK), jnp.int32),
      pltpu.VMEM((CHUNK,), jnp.int32),
      pltpu.VMEM((CHUNK,), jnp.int32),
      pltpu.VMEM((CHUNK,), jnp.int32),
      pltpu.VMEM((CHUNK, D), jnp.float32),
      pltpu.VMEM((CHUNK, D), jnp.float32),
      pltpu.VMEM((CHUNK, D), jnp.float32),
      pltpu.VMEM_SHARED((N_NODES, D), jnp.float32),
      pltpu.SemaphoreType.DMA,
      pltpu.SemaphoreType.DMA,
      pltpu.SemaphoreType.DMA,
      pltpu.SemaphoreType.DMA,
      pltpu.SemaphoreType.DMA,
      pltpu.SemaphoreType.DMA,
      pltpu.SemaphoreType.DMA,
      pltpu.SemaphoreType.DMA,
      pltpu.SemaphoreType.DMA,
  ]

  def body(h_hbm, ep_hbm, agg_out, *rest):
    if with_cnt:
      (cnt_out, sd0, sd1, sd2, db0, db1, db2, r0v, r1v, r2v, acc_sh,
       is0, is1, is2, gs0, gs1, gs2, ss0, ss1, ss2) = rest
    else:
      (sd0, sd1, sd2, db0, db1, db2, r0v, r1v, r2v, acc_sh,
       is0, is1, is2, gs0, gs1, gs2, ss0, ss1, ss2) = rest
    sds = (sd0, sd1, sd2)
    dbs = (db0, db1, db2)
    rvs = (r0v, r1v, r2v)
    iss = (is0, is1, is2)
    gss = (gs0, gs1, gs2)
    sss = (ss0, ss1, ss2)

    c = lax.axis_index("c")
    s = lax.axis_index("s")
    row0 = pl.multiple_of(s * ROWS_A, 8)
    out0 = pl.multiple_of(c * N_NODES + s * ROWS_A, 8)
    cbase = c * CHUNKS_PER_CORE + s * CPT
    clast = cbase + CPT - 1
    xtra = jnp.where(s < NLEFT, 1, 0)
    cx = c * CHUNKS_PER_CORE + NS * CPT + s

    def idx_load(jj, sd, sem):
      pltpu.async_copy(ep_hbm.at[jj], sd, sem)

    def idx_wait(sd, sem):
      pltpu.make_async_copy(ep_hbm.at[0], sd, sem).wait()

    def g_start(sd, rv, sem):
      pltpu.async_copy(h_hbm.at[sd.at[0]], rv, sem)

    def g_wait(sd, rv, sem):
      pltpu.make_async_copy(h_hbm.at[sd.at[0]], rv, sem).wait()

    def copy_dst(sd, db):
      for k in range(CHUNK // 16):
        db[pl.ds(k * 16, 16)] = sd[1, pl.ds(k * 16, 16)]

    def ss_start(rv, db, sem):
      pltpu.async_copy(rv, acc_sh.at[db], sem, add=True)

    def ss_wait(rv, db, sem):
      pltpu.make_async_copy(rv, acc_sh.at[db], sem).wait()

    def zero_stripe(nrows):
      nfull = nrows // CHUNK
      rem = nrows - nfull * CHUNK

      def zinit(j, carry):
        off = pl.multiple_of(row0 + j * CHUNK, 8)
        pltpu.async_copy(r0v, acc_sh.at[pl.ds(off, CHUNK)], gs0)
        return carry

      lax.fori_loop(0, nfull, zinit, 0)
      offr = pl.multiple_of(row0 + nfull * CHUNK, 8)
      pltpu.async_copy(r0v.at[pl.ds(0, rem)], acc_sh.at[pl.ds(offr, rem)],
                       gs0)

      def zdrain(j, carry):
        pltpu.make_async_copy(r0v, acc_sh.at[pl.ds(row0, CHUNK)], gs0).wait()
        return carry

      lax.fori_loop(0, nfull, zdrain, 0)
      pltpu.make_async_copy(r0v.at[pl.ds(0, rem)],
                            acc_sh.at[pl.ds(row0, rem)], gs0).wait()

    def zero_acc():
      @pl.when(s < NS - 1)
      def _():
        zero_stripe(ROWS_A)

      @pl.when(s == NS - 1)
      def _():
        zero_stripe(ROWS_B)

    def copy_out(dst):
      @pl.when(s < NS - 1)
      def _():
        pltpu.sync_copy(acc_sh.at[pl.ds(row0, ROWS_A)],
                        dst.at[pl.ds(out0, ROWS_A)])

      @pl.when(s == NS - 1)
      def _():
        pltpu.sync_copy(acc_sh.at[pl.ds(row0, ROWS_B)],
                        dst.at[pl.ds(out0, ROWS_B)])

    if with_cnt:
      _fill_vmem_rows(r0v, CHUNK, D, 0.0)
      zero_acc()
      _fill_vmem_rows(r0v, CHUNK, D, 1.0)
      plsc.subcore_barrier()

      for i in range(3):
        idx_load(cbase + i, sds[i], iss[i])
      for i in range(3):
        idx_wait(sds[i], iss[i])
        copy_dst(sds[i], dbs[i])
        idx_load(cbase + i + 3, sds[i], iss[i])
        ss_start(r0v, dbs[i], sss[i])

      def ctriple(p, carry):
        j = cbase + 3 + 3 * p
        for i in range(3):
          ss_wait(r0v, dbs[i], sss[i])
          idx_wait(sds[i], iss[i])
          copy_dst(sds[i], dbs[i])
          idx_load(jnp.minimum(j + i + 3, clast), sds[i], iss[i])
          ss_start(r0v, dbs[i], sss[i])
        return carry

      lax.fori_loop(0, CPT // 3 - 1, ctriple, 0)
      for i in range(3):
        ss_wait(r0v, dbs[i], sss[i])
        idx_wait(sds[i], iss[i])

      def cleft(j, carry):
        idx_load(cx, sd0, is0)
        idx_wait(sd0, is0)
        pltpu.sync_copy(r0v, acc_sh.at[sd0.at[1]], add=True)
        return carry

      lax.fori_loop(0, xtra, cleft, 0)
      plsc.subcore_barrier()
      copy_out(cnt_out)
      plsc.subcore_barrier()

    _fill_vmem_rows(r0v, CHUNK, D, 0.0)
    zero_acc()
    plsc.subcore_barrier()

    for i in range(3):
      idx_load(cbase + i, sds[i], iss[i])
    idx_wait(sds[0], iss[0])
    g_start(sds[0], rvs[0], gss[0])
    idx_wait(sds[1], iss[1])
    g_start(sds[1], rvs[1], gss[1])
    idx_wait(sds[2], iss[2])
    g_start(sds[2], rvs[2], gss[2])
    g_wait(sds[0], rvs[0], gss[0])
    copy_dst(sds[0], dbs[0])
    idx_load(cbase + 3, sds[0], iss[0])
    ss_start(rvs[0], dbs[0], sss[0])
    ss_wait(rvs[0], dbs[0], sss[0])
    idx_wait(sds[0], iss[0])
    g_start(sds[0], rvs[0], gss[0])
    g_wait(sds[1], rvs[1], gss[1])
    copy_dst(sds[1], dbs[1])
    idx_load(cbase + 4, sds[1], iss[1])
    ss_start(rvs[1], dbs[1], sss[1])
    ss_wait(rvs[1], dbs[1], sss[1])
    idx_wait(sds[1], iss[1])
    g_start(sds[1], rvs[1], gss[1])
    g_wait(sds[2], rvs[2], gss[2])
    copy_dst(sds[2], dbs[2])
    idx_load(cbase + 5, sds[2], iss[2])
    ss_start(rvs[2], dbs[2], sss[2])

    def triple(p, carry):
      j = cbase + 3 + 3 * p
      for i in range(3):
        ip = (i + 2) % 3
        ss_wait(rvs[ip], dbs[ip], sss[ip])
        idx_wait(sds[ip], iss[ip])
        g_start(sds[ip], rvs[ip], gss[ip])
        g_wait(sds[i], rvs[i], gss[i])
        copy_dst(sds[i], dbs[i])
        idx_load(jnp.minimum(j + i + 3, clast), sds[i], iss[i])
        ss_start(rvs[i], dbs[i], sss[i])
      return carry

    lax.fori_loop(0, CPT // 3 - 1, triple, 0)
    ss_wait(rvs[2], dbs[2], sss[2])
    g_wait(sds[0], rvs[0], gss[0])
    g_wait(sds[1], rvs[1], gss[1])
    idx_wait(sds[2], iss[2])

    def aleft(j, carry):
      idx_load(cx, sd0, is0)
      idx_wait(sd0, is0)
      g_start(sd0, r0v, gs0)
      g_wait(sd0, r0v, gs0)
      pltpu.sync_copy(r0v, acc_sh.at[sd0.at[1]], add=True)
      return carry

    lax.fori_loop(0, xtra, aleft, 0)

    plsc.subcore_barrier()
    copy_out(agg_out)

  return pl.kernel(body, out_type=tuple(out_type), mesh=mesh,
                   scratch_types=scratch)


_sc_agg_cnt = _make_sc_agg(True)
_sc_agg = _make_sc_agg(False)


def _make_tc_combine(relu):
  R = 1000

  def body(agg0, agg1, cnt0, cnt1, h, wl, wr, b, out):
    cnt = cnt0[:, 0:1] + cnt1[:, 0:1]
    inv = 1.0 / jnp.maximum(cnt, 1.0)
    agg = (agg0[...] + agg1[...]) * inv
    acc = jnp.dot(agg, wl[...], preferred_element_type=jnp.float32)
    acc = acc + jnp.dot(h[...], wr[...], preferred_element_type=jnp.float32)
    acc = acc + b[...]
    out[...] = jnp.maximum(acc, 0.0) if relu else acc

  row_spec = pl.BlockSpec((R, D), lambda i: (i, 0))
  w_spec = pl.BlockSpec((D, D), lambda i: (0, 0))
  return pl.pallas_call(
      body,
      grid=(N_NODES // R,),
      in_specs=[row_spec, row_spec, row_spec, row_spec, row_spec,
                w_spec, w_spec, pl.BlockSpec((1, D), lambda i: (0, 0))],
      out_specs=row_spec,
      out_shape=jax.ShapeDtypeStruct((N_NODES, D), jnp.float32),
  )


_tc_combine_relu = _make_tc_combine(True)
_tc_combine = _make_tc_combine(False)


def kernel(x, edge_index, W1l, W1r, b1, W2l, W2r, b2, W3l, W3r, b3):
  src = edge_index[0]
  dst = edge_index[1]
  epairs = jnp.stack(
      [src.reshape(NCHUNKS, CHUNK), dst.reshape(NCHUNKS, CHUNK)], axis=1)

  aggp, cntp = _sc_agg_cnt(x, epairs)
  cnt0, cnt1 = cntp[:N_NODES], cntp[N_NODES:]

  def layer(h, aggp, Wl, Wr, b, relu):
    f = _tc_combine_relu if relu else _tc_combine
    return f(aggp[:N_NODES], aggp[N_NODES:], cnt0, cnt1, h,
             Wl, Wr, b.reshape(1, D))

  h1 = layer(x, aggp, W1l, W1r, b1, True)
  aggp2, = _sc_agg(h1, epairs)
  h2 = layer(h1, aggp2, W2l, W2r, b2, True)
  aggp3, = _sc_agg(h2, epairs)
  h3 = layer(h2, aggp3, W3l, W3r, b3, False)
  return h3
